# baseline (speedup 1.0000x reference)
import functools

import jax
import jax.numpy as jnp
from jax import lax
from jax.experimental import pallas as pl
from jax.experimental.pallas import tpu as pltpu
from jax.experimental.pallas import tpu_sc as plsc

VOCAB = 1000000
D = 64
O = 2
B = 4096
S = 200

NT = 32
BPT = B // NT
GSZ = 16 // O
NG = BPT // GSZ
CH = 128
NCH = S * BPT * O // CH


def _tw_body(t_ref, w_ref, o_ref):
    o_ref[...] = jnp.dot(t_ref[...], w_ref[...],
                         preferred_element_type=jnp.float32)


def _table_w(table, W):
    BLK = 4000
    return pl.pallas_call(
        _tw_body,
        grid=(VOCAB // BLK,),
        in_specs=[
            pl.BlockSpec((BLK, D), lambda i: (i, 0)),
            pl.BlockSpec((D, O), lambda i: (0, 0)),
        ],
        out_specs=pl.BlockSpec((BLK, O), lambda i: (i, 0)),
        out_shape=jax.ShapeDtypeStruct((VOCAB, O), jnp.float32),
    )(table, W)


@functools.partial(
    pl.kernel,
    out_type=jax.ShapeDtypeStruct((B * O,), jnp.float32),
    mesh=plsc.VectorSubcoreMesh(core_axis_name="c", subcore_axis_name="s"),
    scratch_types=[
        pltpu.VMEM((NCH, CH), jnp.int32),
        pltpu.VMEM((S * BPT * O,), jnp.float32),
        pltpu.VMEM((BPT * O,), jnp.float32),
        pltpu.VMEM((16,), jnp.float32),
        pltpu.VMEM((BPT * O,), jnp.float32),
        pltpu.SemaphoreType.DMA,
    ],
)
def _sc_pool(idx_hbm, len2_hbm, b16_hbm, tw_hbm, out_hbm,
             idx_v, rows_v, len_v, b_v, out_v, sem):
    wid = lax.axis_index("s") * 2 + lax.axis_index("c")
    obase = wid * (BPT * O)

    pltpu.sync_copy(idx_hbm.at[wid], idx_v)
    pltpu.sync_copy(len2_hbm.at[pl.ds(obase, BPT * O)], len_v)
    pltpu.sync_copy(b16_hbm, b_v)

    def fire(j, carry):
        pltpu.async_copy(tw_hbm.at[idx_v.at[j]],
                         rows_v.at[pl.ds(j * CH, CH)], sem).wait()
        return carry

    lax.fori_loop(0, NCH, fire, 0)

    bvec = b_v[...]
    for g in range(NG):
        base = g * (S * 16)

        def tb(k, acc, base=base):
            for i in range(8):
                acc = acc + rows_v[pl.ds(base + (k * 8 + i) * 16, 16)]
            return acc

        acc = lax.fori_loop(0, S // 8, tb, jnp.zeros((16,), jnp.float32))
        out_v[pl.ds(g * 16, 16)] = acc / len_v[pl.ds(g * 16, 16)] + bvec

    pltpu.sync_copy(out_v, out_hbm.at[pl.ds(obase, BPT * O)])


def kernel(text, text_lengths, table, W, b):
    text = text.astype(jnp.int32)
    tp = text.reshape(NT, NG, GSZ, S).transpose(0, 1, 3, 2)
    idx = (tp[..., None] * O + jnp.arange(O, dtype=jnp.int32))
    idx = idx.reshape(NT, NCH, CH)
    len2 = jnp.repeat(text_lengths.astype(jnp.float32), O)
    b16 = jnp.tile(b.astype(jnp.float32), 16 // O)
    tw = _table_w(table, W).reshape(VOCAB * O)
    out = _sc_pool(idx, len2, b16, tw)
    return out.reshape(B, O)


# speedup vs baseline: 1.0375x; 1.0375x over previous
import functools

import jax
import jax.numpy as jnp
from jax import lax
from jax.experimental import pallas as pl
from jax.experimental.pallas import tpu as pltpu
from jax.experimental.pallas import tpu_sc as plsc

VOCAB = 1000000
D = 64
O = 2
B = 4096
S = 200

NT = 32
BPT = B // NT
TPT = S * BPT
CH = 128
NCH = TPT // CH
SFULL = (S // 16) * 16


def _tw_body(t_ref, w_ref, o0_ref, o1_ref):
    t = t_ref[...]
    w = w_ref[...]
    o0_ref[...] = jnp.sum(t * w[:, 0], axis=1)
    o1_ref[...] = jnp.sum(t * w[:, 1], axis=1)


def _table_w(table, W):
    BLK = 8192
    return pl.pallas_call(
        _tw_body,
        grid=(pl.cdiv(VOCAB, BLK),),
        in_specs=[
            pl.BlockSpec((BLK, D), lambda i: (i, 0)),
            pl.BlockSpec((D, O), lambda i: (0, 0)),
        ],
        out_specs=[
            pl.BlockSpec((BLK,), lambda i: (i,)),
            pl.BlockSpec((BLK,), lambda i: (i,)),
        ],
        out_shape=[
            jax.ShapeDtypeStruct((VOCAB,), jnp.float32),
            jax.ShapeDtypeStruct((VOCAB,), jnp.float32),
        ],
    )(table, W)


@functools.partial(
    pl.kernel,
    out_type=[
        jax.ShapeDtypeStruct((B,), jnp.float32),
        jax.ShapeDtypeStruct((B,), jnp.float32),
    ],
    mesh=plsc.VectorSubcoreMesh(core_axis_name="c", subcore_axis_name="s"),
    scratch_types=[
        pltpu.VMEM((NCH, CH), jnp.int32),
        pltpu.VMEM((TPT + 16,), jnp.float32),
        pltpu.VMEM((TPT + 16,), jnp.float32),
        pltpu.VMEM((BPT,), jnp.float32),
        pltpu.VMEM((32,), jnp.float32),
        pltpu.VMEM((BPT,), jnp.float32),
        pltpu.VMEM((BPT,), jnp.float32),
        pltpu.SemaphoreType.DMA,
    ],
)
def _sc_pool(text_hbm, len_hbm, b16_hbm, tw0_hbm, tw1_hbm,
             out0_hbm, out1_hbm,
             text_v, r0_v, r1_v, len_v, b_v, o0_v, o1_v, sem):
    wid = lax.axis_index("s") * 2 + lax.axis_index("c")
    bbase = wid * BPT

    pltpu.sync_copy(text_hbm.at[wid], text_v)
    pltpu.sync_copy(len_hbm.at[pl.ds(bbase, BPT)], len_v)
    pltpu.sync_copy(b16_hbm, b_v)

    def chunk_copies(j):
        c0 = pltpu.make_async_copy(tw0_hbm.at[text_v.at[j]],
                                   r0_v.at[pl.ds(j * CH, CH)], sem)
        c1 = pltpu.make_async_copy(tw1_hbm.at[text_v.at[j]],
                                   r1_v.at[pl.ds(j * CH, CH)], sem)
        return c0, c1

    def fire(j, carry):
        c0, c1 = chunk_copies(j)
        c0.start()
        c1.start()

        @pl.when(j > 0)
        def _():
            p0, p1 = chunk_copies(j - 1)
            p0.wait()
            p1.wait()

        return carry

    lax.fori_loop(0, NCH, fire, 0)
    l0, l1 = chunk_copies(NCH - 1)
    l0.wait()
    l1.wait()

    lanes = lax.iota(jnp.int32, 16)
    tail_mask = jnp.where(lanes < (S - SFULL), 1.0, 0.0)
    eqs = [lanes == l for l in range(16)]
    perms = [lanes ^ (1 << k) for k in range(4)]
    b0vec = b_v[pl.ds(0, 16)]
    b1vec = b_v[pl.ds(16, 16)]
    zero = jnp.zeros((16,), jnp.float32)

    gdn = lax.GatherDimensionNumbers(offset_dims=(), collapsed_slice_dims=(0,),
                                     start_index_map=(0,))

    def lane_perm(a, p):
        return lax.gather(a, p[:, None], gdn, (1,),
                          mode=lax.GatherScatterMode.PROMISE_IN_BOUNDS)

    def lane_sum(a):
        for p in perms:
            a = a + lane_perm(a, p)
        return a

    def grp(g, carry):
        gbase = g * 16 * S
        out0 = zero
        out1 = zero
        for l in range(16):
            base = gbase + l * S
            a0 = r0_v[pl.ds(base + SFULL, 16)] * tail_mask
            a1 = r1_v[pl.ds(base + SFULL, 16)] * tail_mask
            for k in range(SFULL // 16):
                a0 = a0 + r0_v[pl.ds(base + k * 16, 16)]
                a1 = a1 + r1_v[pl.ds(base + k * 16, 16)]
            out0 = jnp.where(eqs[l], lane_sum(a0), out0)
            out1 = jnp.where(eqs[l], lane_sum(a1), out1)
        lenvec = len_v[pl.ds(g * 16, 16)]
        o0_v[pl.ds(g * 16, 16)] = out0 * lenvec + b0vec
        o1_v[pl.ds(g * 16, 16)] = out1 * lenvec + b1vec
        return carry

    lax.fori_loop(0, BPT // 16, grp, 0)

    pltpu.sync_copy(o0_v, out0_hbm.at[pl.ds(bbase, BPT)])
    pltpu.sync_copy(o1_v, out1_hbm.at[pl.ds(bbase, BPT)])


def kernel(text, text_lengths, table, W, b):
    text3 = text.astype(jnp.int32).reshape(NT, NCH, CH)
    inv_len = 1.0 / text_lengths.astype(jnp.float32)
    b16 = jnp.repeat(b.astype(jnp.float32), 16)
    tw0, tw1 = _table_w(table, W)
    out0, out1 = _sc_pool(text3, inv_len, b16, tw0, tw1)
    return jnp.stack([out0, out1], axis=1)


# speedup vs baseline: 2.0578x; 1.9834x over previous
import functools

import jax
import jax.numpy as jnp
from jax import lax
from jax.experimental import pallas as pl
from jax.experimental.pallas import tpu as pltpu
from jax.experimental.pallas import tpu_sc as plsc

VOCAB = 1000000
D = 64
O = 2
B = 4096
S = 200

NT = 32
BPT = B // NT
TPT = S * BPT
CH = 128
NCH = TPT // CH


def _tw_body(w_ref, t_ref, o0_ref, o1_ref):
    res = lax.dot_general(w_ref[...], t_ref[...], (((0,), (1,)), ((), ())),
                          preferred_element_type=jnp.float32)
    o0_ref[...] = res[0, :]
    o1_ref[...] = res[1, :]


def _table_w(table, W):
    BLK = 8192
    return pl.pallas_call(
        _tw_body,
        grid=(pl.cdiv(VOCAB, BLK),),
        in_specs=[
            pl.BlockSpec((D, O), lambda i: (0, 0)),
            pl.BlockSpec((BLK, D), lambda i: (i, 0)),
        ],
        out_specs=[
            pl.BlockSpec((BLK,), lambda i: (i,)),
            pl.BlockSpec((BLK,), lambda i: (i,)),
        ],
        out_shape=[
            jax.ShapeDtypeStruct((VOCAB,), jnp.float32),
            jax.ShapeDtypeStruct((VOCAB,), jnp.float32),
        ],
    )(W, table)


def _xp_body(t_ref, o_ref):
    o_ref[...] = t_ref[...].T


def _transpose_text(text):
    return pl.pallas_call(
        _xp_body,
        grid=(NT,),
        in_specs=[pl.BlockSpec((BPT, S), lambda i: (i, 0))],
        out_specs=pl.BlockSpec((S, BPT), lambda i: (0, i)),
        out_shape=jax.ShapeDtypeStruct((S, B), jnp.int32),
    )(text)


@functools.partial(
    pl.kernel,
    out_type=[
        jax.ShapeDtypeStruct((B,), jnp.float32),
        jax.ShapeDtypeStruct((B,), jnp.float32),
    ],
    mesh=plsc.VectorSubcoreMesh(core_axis_name="c", subcore_axis_name="s"),
    scratch_types=[
        pltpu.VMEM((NCH, CH), jnp.int32),
        pltpu.VMEM((TPT,), jnp.float32),
        pltpu.VMEM((TPT,), jnp.float32),
        pltpu.VMEM((BPT,), jnp.float32),
        pltpu.VMEM((32,), jnp.float32),
        pltpu.VMEM((BPT,), jnp.float32),
        pltpu.VMEM((BPT,), jnp.float32),
        pltpu.SemaphoreType.DMA,
    ],
)
def _sc_pool(textt_hbm, len_hbm, b16_hbm, tw0_hbm, tw1_hbm,
             out0_hbm, out1_hbm,
             text_v, r0_v, r1_v, len_v, b_v, o0_v, o1_v, sem):
    wid = lax.axis_index("s") * 2 + lax.axis_index("c")
    bbase = wid * BPT

    pltpu.sync_copy(textt_hbm.at[:, pl.ds(bbase, BPT)], text_v)
    pltpu.sync_copy(len_hbm.at[pl.ds(bbase, BPT)], len_v)
    pltpu.sync_copy(b16_hbm, b_v)

    def chunk_copies(j):
        c0 = pltpu.make_async_copy(tw0_hbm.at[text_v.at[j]],
                                   r0_v.at[pl.ds(j * CH, CH)], sem)
        c1 = pltpu.make_async_copy(tw1_hbm.at[text_v.at[j]],
                                   r1_v.at[pl.ds(j * CH, CH)], sem)
        return c0, c1

    def fire(j, carry):
        c0, c1 = chunk_copies(j)
        c0.start()
        c1.start()

        @pl.when(j > 0)
        def _():
            p0, p1 = chunk_copies(j - 1)
            p0.wait()
            p1.wait()

        return carry

    lax.fori_loop(0, NCH, fire, 0)
    l0, l1 = chunk_copies(NCH - 1)
    l0.wait()
    l1.wait()

    b0vec = b_v[pl.ds(0, 16)]
    b1vec = b_v[pl.ds(16, 16)]
    zero = jnp.zeros((16,), jnp.float32)
    for c in range(BPT // 16):
        coff = c * 16

        def tb(k, accs, coff=coff):
            a0, a1 = accs
            for i in range(8):
                off = (k * 8 + i) * CH + coff
                a0 = a0 + r0_v[pl.ds(off, 16)]
                a1 = a1 + r1_v[pl.ds(off, 16)]
            return a0, a1

        a0, a1 = lax.fori_loop(0, S // 8, tb, (zero, zero))
        lenvec = len_v[pl.ds(coff, 16)]
        o0_v[pl.ds(coff, 16)] = a0 * lenvec + b0vec
        o1_v[pl.ds(coff, 16)] = a1 * lenvec + b1vec

    pltpu.sync_copy(o0_v, out0_hbm.at[pl.ds(bbase, BPT)])
    pltpu.sync_copy(o1_v, out1_hbm.at[pl.ds(bbase, BPT)])


def kernel(text, text_lengths, table, W, b):
    textt = _transpose_text(text.astype(jnp.int32))
    inv_len = 1.0 / text_lengths.astype(jnp.float32)
    b16 = jnp.repeat(b.astype(jnp.float32), 16)
    tw0, tw1 = _table_w(table, W)
    out0, out1 = _sc_pool(textt, inv_len, b16, tw0, tw1)
    return jnp.stack([out0, out1], axis=1)
